# Initial kernel scaffold; baseline (speedup 1.0000x reference)
import jax
import jax.numpy as jnp
from jax.experimental import pallas as pl


def kernel(x, token_table, pos_table):
    raise NotImplementedError("write your pallas kernel here")


# speedup vs baseline: 2.6755x; 2.6755x over previous
import functools

import jax
import jax.numpy as jnp
from jax import lax
from jax.experimental import pallas as pl
from jax.experimental.pallas import tpu as pltpu
from jax.experimental.pallas import tpu_sc as plsc

START = 1
NC = 2
NS = 16
LANES = 16


def _build(B, L, V, E, MAXLEN):
    NW = NC * NS
    BL = B * L
    CH = 2 * L
    IDXM = L // 2
    GQ = CH // IDXM
    NCHUNK = BL // (NW * CH)
    assert BL == NW * CH * NCHUNK and E % LANES == 0 and IDXM <= 128

    mesh = plsc.VectorSubcoreMesh(core_axis_name="c", subcore_axis_name="s")

    @functools.partial(
        pl.kernel,
        out_type=jax.ShapeDtypeStruct((BL, E), jnp.float32),
        mesh=mesh,
        compiler_params=pltpu.CompilerParams(use_tc_tiling_on_sc=False),
        scratch_types=[
            pltpu.VMEM((MAXLEN, E), jnp.float32),
            pltpu.VMEM((2, GQ, IDXM), jnp.int32),
            pltpu.VMEM((2, CH, E), jnp.float32),
            pltpu.SemaphoreType.DMA,
            pltpu.SemaphoreType.DMA,
        ],
    )
    def emb(x2d, token_hbm, pos_hbm, out_hbm, pos_v, idx_v, rows_v, sem0, sem1):
        wid = lax.axis_index("s") * NC + lax.axis_index("c")
        x_base = wid * (NCHUNK * GQ)
        out_base = wid * (NCHUNK * CH)
        sems = (sem0, sem1)

        pltpu.sync_copy(pos_hbm, pos_v)

        def issue(cc, b):
            pltpu.sync_copy(x2d.at[pl.ds(x_base + cc * GQ, GQ)], idx_v.at[b])
            for q in range(GQ):
                pltpu.async_copy(
                    token_hbm.at[idx_v.at[b, q]],
                    rows_v.at[b, pl.ds(q * IDXM, IDXM)],
                    sems[b],
                )

        def drain(b):
            for q in range(GQ):
                pltpu.make_async_copy(
                    token_hbm.at[idx_v.at[b, q]],
                    rows_v.at[b, pl.ds(q * IDXM, IDXM)],
                    sems[b],
                ).wait()

        issue(0, 0)

        def chunk_pair(c2, carry):
            for b in range(2):
                cc = 2 * c2 + b

                @pl.when(cc + 1 < NCHUNK)
                def _():
                    issue(cc + 1, 1 - b)

                drain(b)

                for half in range(CH // L):

                    def add_pos(r, acc, half=half):
                        for j in range(E // LANES):
                            sl = pl.ds(j * LANES, LANES)
                            plsc.addupdate(
                                rows_v.at[b, half * L + r, sl],
                                pos_v[r + START, sl],
                            )
                        return acc

                    lax.fori_loop(0, L, add_pos, 0)
                pltpu.sync_copy(
                    rows_v.at[b],
                    out_hbm.at[pl.ds(out_base + cc * CH, CH)],
                )
            return carry

        lax.fori_loop(0, NCHUNK // 2, chunk_pair, 0)

    return emb


def kernel(x, token_table, pos_table):
    B, L = x.shape
    V, E = token_table.shape
    emb = _build(B, L, V, E, pos_table.shape[0])
    x2d = x.reshape(-1, L // 2)
    out = emb(x2d, token_table, pos_table)
    return out.reshape(B, L, E)


# speedup vs baseline: 2.6835x; 1.0030x over previous
import functools

import jax
import jax.numpy as jnp
from jax import lax
from jax.experimental import pallas as pl
from jax.experimental.pallas import tpu as pltpu
from jax.experimental.pallas import tpu_sc as plsc

START = 1
NC = 2
NS = 16
LANES = 16
RPC = 2


def _build(B, L, V, E, MAXLEN):
    NW = NC * NS
    HL = L // 2
    NCHUNK = B // (NW * RPC)
    assert B == NW * RPC * NCHUNK and E % LANES == 0 and L % 2 == 0 and HL <= 128

    mesh = plsc.VectorSubcoreMesh(core_axis_name="c", subcore_axis_name="s")

    @functools.partial(
        pl.kernel,
        out_type=jax.ShapeDtypeStruct((B, L, E), jnp.float32),
        mesh=mesh,
        compiler_params=pltpu.CompilerParams(use_tc_tiling_on_sc=False),
        scratch_types=[
            pltpu.VMEM((MAXLEN, E), jnp.float32),
            pltpu.VMEM((2, RPC, L), jnp.int32),
            pltpu.VMEM((2, RPC, L, E), jnp.float32),
            pltpu.SemaphoreType.DMA,
            pltpu.SemaphoreType.DMA,
        ],
    )
    def emb(x_hbm, token_hbm, pos_hbm, out_hbm, pos_v, idx_v, rows_v, sem0, sem1):
        wid = lax.axis_index("s") * NC + lax.axis_index("c")
        row_base = wid * (NCHUNK * RPC)
        sems = (sem0, sem1)

        pltpu.sync_copy(pos_hbm, pos_v)

        def copies(cc, b):
            row0 = row_base + cc * RPC
            out = []
            for r in range(RPC):
                out.append((
                    token_hbm.at[idx_v.at[b, r]],
                    rows_v.at[b, r],
                ))
            return row0, out

        def issue(cc, b):
            row0, cps = copies(cc, b)
            pltpu.sync_copy(x_hbm.at[pl.ds(row0, RPC)], idx_v.at[b])
            for src, dst in cps:
                pltpu.async_copy(src, dst, sems[b])

        def drain(cc, b):
            _, cps = copies(cc, b)
            for src, dst in cps:
                pltpu.make_async_copy(src, dst, sems[b]).wait()

        issue(0, 0)

        def chunk_pair(c2, carry):
            for b in range(2):
                cc = 2 * c2 + b

                @pl.when(cc + 1 < NCHUNK)
                def _():
                    issue(cc + 1, 1 - b)

                drain(cc, b)

                for r in range(RPC):

                    def add_pos(l, acc, r=r):
                        for j in range(E // LANES):
                            sl = pl.ds(j * LANES, LANES)
                            plsc.addupdate(
                                rows_v.at[b, r, l, sl],
                                pos_v[l + START, sl],
                            )
                        return acc

                    lax.fori_loop(0, L, add_pos, 0)

                pltpu.sync_copy(
                    rows_v.at[b],
                    out_hbm.at[pl.ds(row_base + cc * RPC, RPC)],
                )
            return carry

        lax.fori_loop(0, NCHUNK // 2, chunk_pair, 0)

    return emb


def kernel(x, token_table, pos_table):
    B, L = x.shape
    V, E = token_table.shape
    emb = _build(B, L, V, E, pos_table.shape[0])
    return emb(x, token_table, pos_table)


# speedup vs baseline: 2.7642x; 1.0301x over previous
import functools

import jax
import jax.numpy as jnp
from jax import lax
from jax.experimental import pallas as pl
from jax.experimental.pallas import tpu as pltpu
from jax.experimental.pallas import tpu_sc as plsc

START = 1
NC = 2
NS = 16
LANES = 16
RPC = 1
NBUF = 4
UNROLL = 8


def _build(B, L, V, E, MAXLEN):
    NW = NC * NS
    BPW = B // NW
    NCHUNK = BPW // RPC
    assert B == NW * BPW and NCHUNK % NBUF == 0
    assert E % LANES == 0 and L % UNROLL == 0 and L <= 256

    mesh = plsc.VectorSubcoreMesh(core_axis_name="c", subcore_axis_name="s")

    @functools.partial(
        pl.kernel,
        out_type=jax.ShapeDtypeStruct((B, L, E), jnp.float32),
        mesh=mesh,
        compiler_params=pltpu.CompilerParams(use_tc_tiling_on_sc=False),
        scratch_types=[
            pltpu.VMEM((MAXLEN, E), jnp.float32),
            pltpu.VMEM((BPW, L), jnp.int32),
            pltpu.VMEM((NBUF, RPC, L, E), jnp.float32),
            [pltpu.SemaphoreType.DMA] * NBUF,
            [pltpu.SemaphoreType.DMA] * NBUF,
        ],
    )
    def emb(x_hbm, token_hbm, pos_hbm, out_hbm, pos_v, idx_v, rows_v, sems, osems):
        wid = lax.axis_index("s") * NC + lax.axis_index("c")
        row_base = wid * BPW

        pltpu.sync_copy(pos_hbm, pos_v)
        pltpu.sync_copy(x_hbm.at[pl.ds(row_base, BPW)], idx_v)

        def gathers(cc, b):
            out = []
            for r in range(RPC):
                out.append((
                    token_hbm.at[idx_v.at[cc * RPC + r]],
                    rows_v.at[b, r],
                ))
            return out

        def out_copy(cc, b):
            return (rows_v.at[b], out_hbm.at[pl.ds(row_base + cc * RPC, RPC)])

        def issue(cc, b):
            for src, dst in gathers(cc, b):
                pltpu.async_copy(src, dst, sems[b])

        def drain(cc, b):
            for src, dst in gathers(cc, b):
                pltpu.make_async_copy(src, dst, sems[b]).wait()

        for p in range(NBUF - 1):
            issue(p, p)

        def chunk_group(cg, carry):
            for b in range(NBUF):
                cc = NBUF * cg + b
                bn = (b + NBUF - 1) % NBUF

                @pl.when(cc + NBUF - 1 < NCHUNK)
                def _():
                    @pl.when(cc >= 1)
                    def _():
                        src, dst = out_copy(cc - 1, bn)
                        pltpu.make_async_copy(src, dst, osems[bn]).wait()

                    issue(cc + NBUF - 1, bn)

                drain(cc, b)

                for r in range(RPC):

                    def add_pos(i, acc, r=r):
                        base_l = i * UNROLL
                        for u in range(UNROLL):
                            lrow = base_l + u
                            for j in range(E // LANES):
                                sl = pl.ds(j * LANES, LANES)
                                plsc.addupdate(
                                    rows_v.at[b, r, lrow, sl],
                                    pos_v[lrow + START, sl],
                                )
                        return acc

                    lax.fori_loop(0, L // UNROLL, add_pos, 0)

                src, dst = out_copy(cc, b)
                pltpu.async_copy(src, dst, osems[b])
            return carry

        lax.fori_loop(0, NCHUNK // NBUF, chunk_group, 0)

        for b in range(NBUF):
            src, dst = out_copy(NCHUNK - NBUF + b, b)
            pltpu.make_async_copy(src, dst, osems[b]).wait()

    return emb


def kernel(x, token_table, pos_table):
    B, L = x.shape
    V, E = token_table.shape
    emb = _build(B, L, V, E, pos_table.shape[0])
    return emb(x, token_table, pos_table)
